# baseline (speedup 1.0000x reference)
import functools

import jax
import jax.numpy as jnp
from jax import lax
from jax.experimental import pallas as pl
from jax.experimental.pallas import tpu as pltpu
from jax.experimental.pallas import tpu_sc as plsc

_NC = 2
_NS = 16
_NW = _NC * _NS
_P = 16
_NB = 5


def _embed(ids, wte, wpe, *, n, d, seq, nbatch):
    span = seq // _NW
    npsub = span // _P
    nsteps = npsub * nbatch

    mesh = plsc.VectorSubcoreMesh(
        core_axis_name="c", subcore_axis_name="s",
        num_cores=_NC, num_subcores=_NS)

    @functools.partial(
        pl.kernel,
        out_type=jax.ShapeDtypeStruct((n, d), jnp.float32),
        mesh=mesh,
        scratch_types=[
            pltpu.VMEM((nbatch, span), jnp.int32),
            pltpu.VMEM((_NB, _P, d), jnp.float32),
            pltpu.VMEM((2, _P, d), jnp.float32),
            pltpu.SemaphoreType.DMA((_NB,)),
            pltpu.SemaphoreType.DMA((_NB,)),
            pltpu.SemaphoreType.DMA((2,)),
        ],
    )
    def run(ids_hbm, wte_hbm, wpe_hbm, out_hbm, idx_v, gbuf, wbuf, gsem, ssem, wsem):
        wid = lax.axis_index("s") * _NC + lax.axis_index("c")
        s0 = wid * span

        pltpu.sync_copy(ids_hbm.at[:, pl.ds(s0, span)], idx_v)

        def g_copy(i):
            p, b, be = i // nbatch, i % nbatch, i % _NB
            return pltpu.make_async_copy(
                wte_hbm.at[idx_v.at[b, pl.ds(p * _P, _P)]],
                gbuf.at[be], gsem.at[be])

        def s_copy(i):
            p, b, be = i // nbatch, i % nbatch, i % _NB
            row0 = b * seq + s0 + p * _P
            return pltpu.make_async_copy(
                gbuf.at[be], out_hbm.at[pl.ds(row0, _P)], ssem.at[be])

        def w_copy(pp):
            return pltpu.make_async_copy(
                wpe_hbm.at[pl.ds(s0 + pp * _P, _P)], wbuf.at[pp % 2],
                wsem.at[pp % 2])

        w_copy(0).start()
        w_copy(1).start()
        g_copy(0).start()
        g_copy(1).start()
        g_copy(2).start()

        def step(i, carry):
            p = i // nbatch
            b = i % nbatch
            be = i % _NB

            @pl.when(i >= 2)
            def _():
                s_copy(i - 2).wait()

            @pl.when(i + 3 < nsteps)
            def _():
                g_copy(i + 3).start()

            g_copy(i).wait()

            @pl.when(b == 0)
            def _():
                w_copy(p).wait()

            wsel = p % 2
            nvec = d // 16

            @plsc.parallel_loop(0, _P * nvec, 1, unroll=16)
            def add_body(v):
                r = v // nvec
                sl = pl.ds((v % nvec) * 16, 16)
                plsc.addupdate(gbuf.at[be, r, sl], wbuf[wsel, r, sl])
            s_copy(i).start()

            @pl.when(jnp.logical_and(b == nbatch - 1, p + 2 < npsub))
            def _():
                w_copy(p + 2).start()

            return carry

        lax.fori_loop(0, nsteps, step, 0)
        s_copy(nsteps - 2).wait()
        s_copy(nsteps - 1).wait()

    return run(ids, wte, wpe)


def kernel(input_ids, wte, wpe):
    nbatch, seq = input_ids.shape
    d = wte.shape[1]
    n = nbatch * seq
    out = _embed(input_ids.astype(jnp.int32), wte, wpe,
                 n=n, d=d, seq=seq, nbatch=nbatch)
    return out.reshape(nbatch, seq, d)


# speedup vs baseline: 1.0078x; 1.0078x over previous
import functools

import jax
import jax.numpy as jnp
from jax import lax
from jax.experimental import pallas as pl
from jax.experimental.pallas import tpu as pltpu
from jax.experimental.pallas import tpu_sc as plsc

_NC = 2
_NS = 16
_NW = _NC * _NS
_P = 16
_NB = 4


def _embed(ids, wte, wpe, *, n, d, seq, nbatch):
    span = seq // _NW
    npsub = span // _P
    nsteps = npsub * nbatch

    mesh = plsc.VectorSubcoreMesh(
        core_axis_name="c", subcore_axis_name="s",
        num_cores=_NC, num_subcores=_NS)

    @functools.partial(
        pl.kernel,
        out_type=jax.ShapeDtypeStruct((n, d), jnp.float32),
        mesh=mesh,
        scratch_types=[
            pltpu.VMEM((nbatch, span), jnp.int32),
            pltpu.VMEM((_NB, _P, d), jnp.float32),
            pltpu.VMEM((2, _P, d), jnp.float32),
            pltpu.SemaphoreType.DMA((_NB,)),
            pltpu.SemaphoreType.DMA((_NB,)),
            pltpu.SemaphoreType.DMA((2,)),
        ],
    )
    def run(ids_hbm, wte_hbm, wpe_hbm, out_hbm, idx_v, gbuf, wbuf, gsem, ssem, wsem):
        wid = lax.axis_index("s") * _NC + lax.axis_index("c")
        s0 = wid * span

        pltpu.sync_copy(ids_hbm.at[:, pl.ds(s0, span)], idx_v)

        def g_copy(i):
            p, b, be = i // nbatch, i % nbatch, i % _NB
            return pltpu.make_async_copy(
                wte_hbm.at[idx_v.at[b, pl.ds(p * _P, _P)]],
                gbuf.at[be], gsem.at[be])

        def s_copy(i):
            p, b, be = i // nbatch, i % nbatch, i % _NB
            row0 = b * seq + s0 + p * _P
            return pltpu.make_async_copy(
                gbuf.at[be], out_hbm.at[pl.ds(row0, _P)], ssem.at[be])

        def w_copy(pp):
            return pltpu.make_async_copy(
                wpe_hbm.at[pl.ds(s0 + pp * _P, _P)], wbuf.at[pp % 2],
                wsem.at[pp % 2])

        w_copy(0).start()
        w_copy(1).start()
        g_copy(0).start()
        g_copy(1).start()

        nvec = d // 16

        def step(i, carry):
            p = i // nbatch
            b = i % nbatch
            be = i % _NB

            @pl.when(i >= 2)
            def _():
                s_copy(i - 2).wait()

            @pl.when(i + 2 < nsteps)
            def _():
                g_copy(i + 2).start()

            g_copy(i).wait()

            @pl.when(b == 0)
            def _():
                w_copy(p).wait()

            wsel = p % 2

            @plsc.parallel_loop(0, _P * nvec, 1, unroll=16)
            def add_body(v):
                r = v // nvec
                sl = pl.ds((v % nvec) * 16, 16)
                plsc.addupdate(gbuf.at[be, r, sl], wbuf[wsel, r, sl])

            s_copy(i).start()

            @pl.when(jnp.logical_and(b == nbatch - 1, p + 2 < npsub))
            def _():
                w_copy(p + 2).start()

            return carry

        lax.fori_loop(0, nsteps, step, 0)
        s_copy(nsteps - 2).wait()
        s_copy(nsteps - 1).wait()

    return run(ids, wte, wpe)


def kernel(input_ids, wte, wpe):
    nbatch, seq = input_ids.shape
    d = wte.shape[1]
    n = nbatch * seq
    out = _embed(input_ids.astype(jnp.int32), wte, wpe,
                 n=n, d=d, seq=seq, nbatch=nbatch)
    return out.reshape(nbatch, seq, d)


# speedup vs baseline: 1.0079x; 1.0002x over previous
import functools

import jax
import jax.numpy as jnp
from jax import lax
from jax.experimental import pallas as pl
from jax.experimental.pallas import tpu as pltpu
from jax.experimental.pallas import tpu_sc as plsc

_NC = 2
_NS = 16
_NW = _NC * _NS
_P = 16
_NB = 4


def _embed(ids, wte, wpe, *, n, d, seq, nbatch):
    span = seq // _NW
    npsub = span // _P
    nsteps = npsub * nbatch

    mesh = plsc.VectorSubcoreMesh(
        core_axis_name="c", subcore_axis_name="s",
        num_cores=_NC, num_subcores=_NS)

    @functools.partial(
        pl.kernel,
        out_type=jax.ShapeDtypeStruct((n, d), jnp.float32),
        mesh=mesh,
        scratch_types=[
            pltpu.VMEM((nbatch, span), jnp.int32),
            pltpu.VMEM((_NB, _P, d), jnp.float32),
            pltpu.VMEM((2, _P, d), jnp.float32),
            pltpu.SemaphoreType.DMA((_NB,)),
            pltpu.SemaphoreType.DMA((_NB,)),
            pltpu.SemaphoreType.DMA((2,)),
        ],
    )
    def run(ids_hbm, wte_hbm, wpe_hbm, out_hbm, idx_v, gbuf, wbuf, gsem, ssem, wsem):
        wid = lax.axis_index("s") * _NC + lax.axis_index("c")
        s0 = wid * span

        pltpu.sync_copy(ids_hbm.at[:, pl.ds(s0, span)], idx_v)

        _H = _P // 2

        def g_half(i, h):
            p, b, be = i // nbatch, i % nbatch, i % _NB
            return pltpu.make_async_copy(
                wte_hbm.at[idx_v.at[b, pl.ds(p * _P + h * _H, _H)]],
                gbuf.at[be, pl.ds(h * _H, _H)], gsem.at[be])

        class _GPair:
            def __init__(self, i):
                self.i = i

            def start(self):
                g_half(self.i, 0).start()
                g_half(self.i, 1).start()

            def wait(self):
                g_half(self.i, 0).wait()
                g_half(self.i, 1).wait()

        def g_copy(i):
            return _GPair(i)

        def s_copy(i):
            p, b, be = i // nbatch, i % nbatch, i % _NB
            row0 = b * seq + s0 + p * _P
            return pltpu.make_async_copy(
                gbuf.at[be], out_hbm.at[pl.ds(row0, _P)], ssem.at[be])

        def w_copy(pp):
            return pltpu.make_async_copy(
                wpe_hbm.at[pl.ds(s0 + pp * _P, _P)], wbuf.at[pp % 2],
                wsem.at[pp % 2])

        w_copy(0).start()
        w_copy(1).start()
        g_copy(0).start()
        g_copy(1).start()

        nvec = d // 16

        def step(i, carry):
            p = i // nbatch
            b = i % nbatch
            be = i % _NB

            @pl.when(i >= 2)
            def _():
                s_copy(i - 2).wait()

            @pl.when(i + 2 < nsteps)
            def _():
                g_copy(i + 2).start()

            g_copy(i).wait()

            @pl.when(b == 0)
            def _():
                w_copy(p).wait()

            wsel = p % 2

            @plsc.parallel_loop(0, _P * nvec, 1, unroll=16)
            def add_body(v):
                r = v // nvec
                sl = pl.ds((v % nvec) * 16, 16)
                plsc.addupdate(gbuf.at[be, r, sl], wbuf[wsel, r, sl])

            s_copy(i).start()

            @pl.when(jnp.logical_and(b == nbatch - 1, p + 2 < npsub))
            def _():
                w_copy(p + 2).start()

            return carry

        lax.fori_loop(0, nsteps, step, 0)
        s_copy(nsteps - 2).wait()
        s_copy(nsteps - 1).wait()

    return run(ids, wte, wpe)


def kernel(input_ids, wte, wpe):
    nbatch, seq = input_ids.shape
    d = wte.shape[1]
    n = nbatch * seq
    out = _embed(input_ids.astype(jnp.int32), wte, wpe,
                 n=n, d=d, seq=seq, nbatch=nbatch)
    return out.reshape(nbatch, seq, d)


# speedup vs baseline: 1.0100x; 1.0020x over previous
import functools

import jax
import jax.numpy as jnp
from jax import lax
from jax.experimental import pallas as pl
from jax.experimental.pallas import tpu as pltpu
from jax.experimental.pallas import tpu_sc as plsc

_NC = 2
_NS = 16
_NW = _NC * _NS
_P = 16
_NB = 4


def _embed(ids, wte, wpe, *, n, d, seq, nbatch):
    span = seq // _NW
    npsub = span // _P
    nsteps = npsub * nbatch

    mesh = plsc.VectorSubcoreMesh(
        core_axis_name="c", subcore_axis_name="s",
        num_cores=_NC, num_subcores=_NS)

    @functools.partial(
        pl.kernel,
        out_type=jax.ShapeDtypeStruct((n, d), jnp.float32),
        mesh=mesh,
        scratch_types=[
            pltpu.VMEM((nbatch, span), jnp.int32),
            pltpu.VMEM((_NB, _P, d), jnp.float32),
            pltpu.VMEM((2, _P, d), jnp.float32),
            pltpu.SemaphoreType.DMA((_NB,)),
            pltpu.SemaphoreType.DMA((_NB,)),
            pltpu.SemaphoreType.DMA((2,)),
        ],
    )
    def run(ids_hbm, wte_hbm, wpe_hbm, out_hbm, idx_v, gbuf, wbuf, gsem, ssem, wsem):
        wid = lax.axis_index("s") * _NC + lax.axis_index("c")
        s0 = wid * span

        pltpu.sync_copy(ids_hbm.at[:, pl.ds(s0, span)], idx_v)

        def g_copy(i):
            p, b, be = i // nbatch, i % nbatch, i % _NB
            return pltpu.make_async_copy(
                wte_hbm.at[idx_v.at[b, pl.ds(p * _P, _P)]],
                gbuf.at[be], gsem.at[be])

        def s_copy(i):
            p, b, be = i // nbatch, i % nbatch, i % _NB
            row0 = b * seq + s0 + p * _P
            return pltpu.make_async_copy(
                gbuf.at[be], out_hbm.at[pl.ds(row0, _P)], ssem.at[be])

        def w_copy(pp):
            return pltpu.make_async_copy(
                wpe_hbm.at[pl.ds(s0 + pp * _P, _P)], wbuf.at[pp % 2],
                wsem.at[pp % 2])

        w_copy(0).start()
        w_copy(1).start()
        g_copy(0).start()
        g_copy(1).start()

        nvec = d // 16

        def step(i, carry):
            p = i // nbatch
            b = i % nbatch
            be = i % _NB

            @pl.when(i >= 2)
            def _():
                s_copy(i - 2).wait()

            @pl.when(i + 2 < nsteps)
            def _():
                g_copy(i + 2).start()

            g_copy(i).wait()

            @pl.when(b == 0)
            def _():
                w_copy(p).wait()

            wsel = p % 2

            @plsc.parallel_loop(0, _P * nvec, 1, unroll=16)
            def add_body(v):
                r = v // nvec
                sl = pl.ds((v % nvec) * 16, 16)
                plsc.addupdate(gbuf.at[be, r, sl], wbuf[wsel, r, sl])

            s_copy(i).start()

            @pl.when(jnp.logical_and(b == nbatch - 1, p + 2 < npsub))
            def _():
                w_copy(p + 2).start()

            return carry

        lax.fori_loop(0, nsteps, step, 0)
        s_copy(nsteps - 2).wait()
        s_copy(nsteps - 1).wait()

    return run(ids, wte, wpe)


def kernel(input_ids, wte, wpe):
    nbatch, seq = input_ids.shape
    d = wte.shape[1]
    n = nbatch * seq
    out = _embed(input_ids.astype(jnp.int32), wte, wpe,
                 n=n, d=d, seq=seq, nbatch=nbatch)
    return out.reshape(nbatch, seq, d)
